# baseline (speedup 1.0000x reference)
import functools

import jax
import jax.numpy as jnp
from jax import lax
from jax.experimental import pallas as pl
from jax.experimental.pallas import tpu as pltpu
from jax.experimental.pallas import tpu_sc as plsc

E = 8
RANK = 32
D = 2048
ALPHA = 32.0
BM = 1024

NC = 2
NS = 16
NW = NC * NS
LANES = 16


def _stage_a(h_ref, wt_ref, aq_ref, av_ref, lt_ref, tq_ref, tv_ref):
    h = h_ref[...]
    logits = jnp.dot(h, wt_ref[...], preferred_element_type=jnp.float32)
    lt_ref[...] = logits.T
    tq_ref[...] = jnp.dot(h, aq_ref[...],
                          preferred_element_type=jnp.float32).astype(jnp.bfloat16)
    tv_ref[...] = jnp.dot(h, av_ref[...],
                          preferred_element_type=jnp.float32).astype(jnp.bfloat16)


def _route_body(lt_hbm, idx_hbm, s_hbm, lg_v, idx_v, s_v):
    n_tok = lt_hbm.shape[1]
    per_w = n_tok // NW
    wid = lax.axis_index("s") * NC + lax.axis_index("c")
    base = wid * per_w
    pltpu.sync_copy(lt_hbm.at[:, pl.ds(base, per_w)], lg_v)
    scale = ALPHA / float(RANK)
    for g in range(per_w // LANES):
        off = g * LANES
        m = lg_v[0, pl.ds(off, LANES)]
        bidx = jnp.zeros((LANES,), jnp.int32)
        for e in range(1, E):
            le = lg_v[e, pl.ds(off, LANES)]
            gt = le > m
            m = jnp.where(gt, le, m)
            bidx = jnp.where(gt, e, bidx)
        ssum = jnp.zeros((LANES,), jnp.float32)
        for e in range(E):
            ssum = ssum + jnp.exp(lg_v[e, pl.ds(off, LANES)] - m)
        idx_v[pl.ds(off, LANES)] = bidx
        s_v[pl.ds(off, LANES)] = scale / ssum
    pltpu.sync_copy(idx_v, idx_hbm.at[pl.ds(base, per_w)])
    pltpu.sync_copy(s_v, s_hbm.at[pl.ds(base, per_w)])


def _stage_c(tq_ref, tv_ref, idx_ref, s_ref, bq_ref, bv_ref, q_ref, v_ref):
    idx = idx_ref[...]
    s = s_ref[...]
    col_expert = jax.lax.broadcasted_iota(jnp.int32, (BM, E * RANK), 1) // RANK
    keep = (col_expert == idx).astype(jnp.bfloat16)
    q_ref[...] = jnp.dot(tq_ref[...] * keep, bq_ref[...],
                         preferred_element_type=jnp.float32) * s
    v_ref[...] = jnp.dot(tv_ref[...] * keep, bv_ref[...],
                         preferred_element_type=jnp.float32) * s


@jax.jit
def _run(h, wt, aq, bq, av, bv):
    n_tokens = h.shape[0]
    grid = (n_tokens // BM,)
    full = lambda shape: pl.BlockSpec(shape, lambda i: (0, 0))

    lt, tq, tv = pl.pallas_call(
        _stage_a,
        grid=grid,
        in_specs=[
            pl.BlockSpec((BM, D), lambda i: (i, 0)),
            full((D, E)),
            full((D, E * RANK)),
            full((D, E * RANK)),
        ],
        out_specs=[
            pl.BlockSpec((E, BM), lambda i: (0, i)),
            pl.BlockSpec((BM, E * RANK), lambda i: (i, 0)),
            pl.BlockSpec((BM, E * RANK), lambda i: (i, 0)),
        ],
        out_shape=[
            jax.ShapeDtypeStruct((E, n_tokens), jnp.float32),
            jax.ShapeDtypeStruct((n_tokens, E * RANK), jnp.bfloat16),
            jax.ShapeDtypeStruct((n_tokens, E * RANK), jnp.bfloat16),
        ],
    )(h, wt, aq, av)

    per_w = n_tokens // NW
    route = pl.kernel(
        _route_body,
        out_type=[
            jax.ShapeDtypeStruct((n_tokens,), jnp.int32),
            jax.ShapeDtypeStruct((n_tokens,), jnp.float32),
        ],
        mesh=plsc.VectorSubcoreMesh(core_axis_name="c", subcore_axis_name="s",
                                    num_cores=NC, num_subcores=NS),
        scratch_types=[
            pltpu.VMEM((E, per_w), jnp.float32),
            pltpu.VMEM((per_w,), jnp.int32),
            pltpu.VMEM((per_w,), jnp.float32),
        ],
    )
    idx, s = route(lt)

    q, v = pl.pallas_call(
        _stage_c,
        grid=grid,
        in_specs=[
            pl.BlockSpec((BM, E * RANK), lambda i: (i, 0)),
            pl.BlockSpec((BM, E * RANK), lambda i: (i, 0)),
            pl.BlockSpec((BM, 1), lambda i: (i, 0)),
            pl.BlockSpec((BM, 1), lambda i: (i, 0)),
            full((E * RANK, D)),
            full((E * RANK, D)),
        ],
        out_specs=[
            pl.BlockSpec((BM, D), lambda i: (i, 0)),
            pl.BlockSpec((BM, D), lambda i: (i, 0)),
        ],
        out_shape=[
            jax.ShapeDtypeStruct((n_tokens, D), jnp.float32),
            jax.ShapeDtypeStruct((n_tokens, D), jnp.float32),
        ],
    )(tq, tv, idx.reshape(-1, 1), s.reshape(-1, 1), bq, bv)
    return q, v


def kernel(hidden_states, router_weight, q_lora_a, q_lora_b, v_lora_a, v_lora_b):
    orig_shape = hidden_states.shape[:-1]
    h = hidden_states.reshape(-1, hidden_states.shape[-1]).astype(jnp.bfloat16)
    wt = router_weight.T
    aq = q_lora_a.transpose(1, 0, 2).reshape(D, E * RANK).astype(jnp.bfloat16)
    bq = q_lora_b.reshape(E * RANK, -1).astype(jnp.bfloat16)
    av = v_lora_a.transpose(1, 0, 2).reshape(D, E * RANK).astype(jnp.bfloat16)
    bv = v_lora_b.reshape(E * RANK, -1).astype(jnp.bfloat16)
    q, v = _run(h, wt, aq, bq, av, bv)
    q_out = q_lora_b.shape[-1]
    v_out = v_lora_b.shape[-1]
    return (q.reshape(*orig_shape, q_out), v.reshape(*orig_shape, v_out))


# speedup vs baseline: 1.6672x; 1.6672x over previous
import functools

import jax
import jax.numpy as jnp
from jax.experimental import pallas as pl
from jax.experimental.pallas import tpu as pltpu

E = 8
RANK = 32
D = 2048
ALPHA = 32.0
BM = 512


def _body(h_ref, wt_ref, aq_ref, bq_ref, av_ref, bv_ref, q_ref, v_ref):
    h = h_ref[...]
    logits = jnp.dot(h, wt_ref[...], preferred_element_type=jnp.float32)
    m = jnp.max(logits, axis=1, keepdims=True)
    p = jnp.exp(logits - m)
    score = 1.0 / jnp.sum(p, axis=1, keepdims=True)
    idx = jnp.argmax(logits, axis=1)
    s = score * (ALPHA / float(RANK))

    col_expert = jax.lax.broadcasted_iota(jnp.int32, (BM, E * RANK), 1) // RANK
    keep = (col_expert == idx[:, None]).astype(jnp.bfloat16)

    hb = h.astype(jnp.bfloat16)
    tq = jnp.dot(hb, aq_ref[...], preferred_element_type=jnp.float32)
    tq = tq.astype(jnp.bfloat16) * keep
    q_ref[...] = jnp.dot(tq, bq_ref[...], preferred_element_type=jnp.float32) * s

    tv = jnp.dot(hb, av_ref[...], preferred_element_type=jnp.float32)
    tv = tv.astype(jnp.bfloat16) * keep
    v_ref[...] = jnp.dot(tv, bv_ref[...], preferred_element_type=jnp.float32) * s


@jax.jit
def _run(h, wt, aq, bq, av, bv):
    n_tokens = h.shape[0]
    grid = (n_tokens // BM,)
    full = lambda shape: pl.BlockSpec(shape, lambda i: (0, 0))
    q, v = pl.pallas_call(
        _body,
        grid=grid,
        in_specs=[
            pl.BlockSpec((BM, D), lambda i: (i, 0)),
            full((D, E)),
            full((D, E * RANK)),
            full((E * RANK, D)),
            full((D, E * RANK)),
            full((E * RANK, D)),
        ],
        out_specs=[
            pl.BlockSpec((BM, D), lambda i: (i, 0)),
            pl.BlockSpec((BM, D), lambda i: (i, 0)),
        ],
        out_shape=[
            jax.ShapeDtypeStruct((n_tokens, D), jnp.float32),
            jax.ShapeDtypeStruct((n_tokens, D), jnp.float32),
        ],
    )(h, wt, aq, bq, av, bv)
    return q, v


def kernel(hidden_states, router_weight, q_lora_a, q_lora_b, v_lora_a, v_lora_b):
    orig_shape = hidden_states.shape[:-1]
    h = hidden_states.reshape(-1, hidden_states.shape[-1])
    wt = router_weight.T
    aq = q_lora_a.transpose(1, 0, 2).reshape(D, E * RANK).astype(jnp.bfloat16)
    bq = q_lora_b.reshape(E * RANK, -1).astype(jnp.bfloat16)
    av = v_lora_a.transpose(1, 0, 2).reshape(D, E * RANK).astype(jnp.bfloat16)
    bv = v_lora_b.reshape(E * RANK, -1).astype(jnp.bfloat16)
    q, v = _run(h, wt, aq, bq, av, bv)
    q_out = q_lora_b.shape[-1]
    v_out = v_lora_b.shape[-1]
    return (q.reshape(*orig_shape, q_out), v.reshape(*orig_shape, v_out))
